# baseline (speedup 1.0000x reference)
import jax
import jax.numpy as jnp
from jax.experimental import pallas as pl
from jax.experimental.pallas import tpu as pltpu
from jax.experimental.pallas import tpu_sc as plsc

_NQ = 2048
_K = 3
_BIG = 1e30


def _knn_body(q_ref, kxyzT_ref, bidx_ref, idx_ref, w_ref, *, half):
    q = q_ref[...]
    bq = q.shape[0]
    m = kxyzT_ref.shape[1]
    kT = kxyzT_ref[...]
    cross = jax.lax.dot_general(q, kT, (((1,), (0,)), ((), ())),
                                preferred_element_type=jnp.float32)
    k2 = jnp.sum(kT * kT, axis=0, keepdims=True)
    q2 = jnp.sum(q * q, axis=1, keepdims=True)
    acc = (q2 + k2) - 2.0 * cross
    bid = bidx_ref[...]
    d2 = jnp.where(bid == 0, acc[:, :half], acc[:, half:])
    colid = jax.lax.broadcasted_iota(jnp.int32, (bq, half), 1)
    idxs, vals = [], []
    for k in range(_K):
        mv = jnp.min(d2, axis=1, keepdims=True)
        am = jnp.min(jnp.where(d2 == mv, colid, half), axis=1, keepdims=True)
        idxs.append(am)
        vals.append(mv)
        if k < _K - 1:
            d2 = jnp.where(colid == am, _BIG, d2)
    idx = jnp.concatenate(idxs, axis=1)
    val = jnp.concatenate(vals, axis=1)
    dist = jnp.maximum(val, 1e-10)
    recip = 1.0 / (dist + 1e-8)
    w = recip / jnp.sum(recip, axis=1, keepdims=True)
    idx_ref[...] = idx + bid * half
    w_ref[...] = w


_KNN_BQ = {8192: 128, 2048: 256, 512: 512, 128: 2048, 32: 2048}


def _knn(q, kxyzT, bidx2d, half):
    import functools
    m = kxyzT.shape[1]
    bq = _KNN_BQ[half]
    grid = (_NQ // bq,)
    return pl.pallas_call(
        functools.partial(_knn_body, half=half),
        grid=grid,
        in_specs=[
            pl.BlockSpec((bq, 3), lambda i: (i, 0)),
            pl.BlockSpec((3, m), lambda i: (0, 0)),
            pl.BlockSpec((bq, 1), lambda i: (i, 0)),
        ],
        out_specs=[
            pl.BlockSpec((bq, _K), lambda i: (i, 0)),
            pl.BlockSpec((bq, _K), lambda i: (i, 0)),
        ],
        out_shape=[
            jax.ShapeDtypeStruct((_NQ, _K), jnp.int32),
            jax.ShapeDtypeStruct((_NQ, _K), jnp.float32),
        ],
    )(q, kxyzT, bidx2d)


def _sc_gather(data, idx_flat, csub):
    rows, c = data.shape
    chunks = c // csub
    if chunks > 1:
        data = data.reshape(rows * chunks, csub)
        idx_flat = (idx_flat[0][:, None] * chunks
                    + jnp.arange(chunks, dtype=jnp.int32)).reshape(1, -1)
    window = 128
    num = idx_flat.shape[1]
    c = csub
    mesh = plsc.VectorSubcoreMesh(core_axis_name="c", subcore_axis_name="s")

    @pl.kernel(out_type=jax.ShapeDtypeStruct((num, c), data.dtype), mesh=mesh)
    def k(x_hbm, i_hbm, o_hbm):
        def body(i_vmem, o_vmem):
            pltpu.sync_copy(x_hbm.at[i_vmem.at[0]], o_vmem)

        pltpu.emit_pipeline(
            body,
            grid=(num // window,),
            in_specs=[pl.BlockSpec((1, window), index_map=lambda i: (0, i))],
            out_specs=[pl.BlockSpec((window, c), index_map=lambda i: (i, 0))],
            core_axis_name=("c", "s"),
            dimension_semantics=(pltpu.PARALLEL,),
        )(i_hbm, o_hbm)

    out = k(data, idx_flat)
    if chunks > 1:
        out = out.reshape(num // chunks, chunks * csub)
    return out


def _mlp_body(*refs):
    (w1r, w2r, w3r, w4r, w5r,
     g10, g11, g12, g20, g21, g22,
     i3, i4, i5, t3, t4, t5,
     w1cat, m2, m3, m4, b4r, out_ref) = refs
    wrs = (w1r, w2r, w3r, w4r, w5r)
    grs = ((g10, g11, g12), (g20, g21, g22))
    dn = (((1,), (1,)), ((), ()))
    dn0 = (((1,), (0,)), ((), ()))
    xs = []
    for i in range(2):
        wv = wrs[i][...]
        xs.append(grs[i][0][...].astype(jnp.float32) * wv[:, 0:1]
                  + grs[i][1][...].astype(jnp.float32) * wv[:, 1:2]
                  + grs[i][2][...].astype(jnp.float32) * wv[:, 2:3])
    bq = wrs[0].shape[0]
    for j, (ir, tr) in enumerate(((i3, t3), (i4, t4), (i5, t5))):
        rows = tr.shape[0]
        idxv = ir[...]
        wv = wrs[2 + j][...]
        colid = jax.lax.broadcasted_iota(jnp.int32, (bq, rows), 1)
        s = jnp.where(colid == idxv[:, 0:1], wv[:, 0:1], 0.0)
        s = s + jnp.where(colid == idxv[:, 1:2], wv[:, 1:2], 0.0)
        s = s + jnp.where(colid == idxv[:, 2:3], wv[:, 2:3], 0.0)
        xs.append(jax.lax.dot_general(s, tr[...], dn0,
                                      precision=jax.lax.Precision.HIGHEST,
                                      preferred_element_type=jnp.float32))
    x = jnp.concatenate(xs, axis=1)
    h = jax.lax.dot_general(x, w1cat[...], dn,
                            preferred_element_type=jnp.float32)
    h = jnp.maximum(h, 0.0)
    h = jnp.maximum(jax.lax.dot_general(h, m2[...], dn,
                                        preferred_element_type=jnp.float32), 0.0)
    h = jnp.maximum(jax.lax.dot_general(h, m3[...], dn,
                                        preferred_element_type=jnp.float32), 0.0)
    out_ref[...] = (jax.lax.dot_general(h, m4[...], dn,
                                        preferred_element_type=jnp.float32)
                    + b4r[...])


def _mlp(ws, gs, idxs, tabs, w1cat, w2, w3, w4, b4_2d, n_classes):
    bq = 256
    grid = (_NQ // bq,)
    in_arrays = list(ws)
    in_specs = [pl.BlockSpec((bq, _K), lambda i: (i, 0)) for _ in ws]
    for trip in gs:
        for g in trip:
            in_arrays.append(g)
            in_specs.append(pl.BlockSpec((bq, g.shape[1]), lambda i: (i, 0)))
    for ix in idxs:
        in_arrays.append(ix)
        in_specs.append(pl.BlockSpec((bq, _K), lambda i: (i, 0)))
    for t in tabs:
        in_arrays.append(t)
        in_specs.append(pl.BlockSpec(t.shape, lambda i: (0, 0)))
    for mmat in (w1cat, w2, w3, w4, b4_2d):
        in_arrays.append(mmat)
        in_specs.append(pl.BlockSpec(mmat.shape, lambda i: (0, 0)))
    return pl.pallas_call(
        _mlp_body,
        grid=grid,
        in_specs=in_specs,
        out_specs=pl.BlockSpec((bq, n_classes), lambda i: (i, 0)),
        out_shape=jax.ShapeDtypeStruct((_NQ, n_classes), jnp.float32),
    )(*in_arrays)


def kernel(weakly_points, res1_xyz, res1_features, res2_xyz, res2_features,
           res3_xyz, res3_features, res4_xyz, res4_features, res5_xyz,
           res5_features, batch_inds, W1, W2, W3, W4, b4):
    q = weakly_points
    bidx2d = batch_inds.reshape(_NQ, 1)
    stages = [(res1_xyz, res1_features), (res2_xyz, res2_features),
              (res3_xyz, res3_features), (res4_xyz, res4_features),
              (res5_xyz, res5_features)]
    ws, gs, idxs, tabs, w1_parts = [], [], [], [], []
    off = 0
    for si, (xyz, feat) in enumerate(stages):
        b, n, _ = xyz.shape
        c = feat.shape[1]
        kxyzT = xyz.reshape(b * n, 3).T
        idx, w = _knn(q, kxyzT, bidx2d, half=n)
        ws.append(w)
        featT = feat.transpose(0, 2, 1).reshape(b * n, c)
        cp = -(-c // 256) * 256 if si < 2 else -(-c // 128) * 128
        if cp != c:
            featT = jnp.pad(featT, ((0, 0), (0, cp - c)))
        if si < 2:
            idx_flat = idx.T.reshape(1, _K * _NQ)
            packed = jax.lax.bitcast_convert_type(
                featT.astype(jnp.bfloat16).reshape(b * n, cp // 2, 2),
                jnp.int32)
            gath_i = _sc_gather(packed, idx_flat, cp // 2)
            gath = jax.lax.bitcast_convert_type(
                gath_i, jnp.bfloat16).reshape(_K * _NQ, cp)
            g3 = gath.reshape(_K, _NQ, cp)
            gs.append((g3[0], g3[1], g3[2]))
        else:
            idxs.append(idx)
            tabs.append(featT)
        w1p = W1[:, off:off + c]
        if cp != c:
            w1p = jnp.pad(w1p, ((0, 0), (0, cp - c)))
        w1_parts.append(w1p)
        off += c
    n_classes = W4.shape[0]
    b4_2d = b4.reshape(1, n_classes)
    w1cat = jnp.concatenate(w1_parts, axis=1)
    return _mlp(ws, gs, idxs, tabs, w1cat, W2, W3, W4, b4_2d, n_classes)


# speedup vs baseline: 1.5293x; 1.5293x over previous
import jax
import jax.numpy as jnp
from jax.experimental import pallas as pl
from jax.experimental.pallas import tpu as pltpu
from jax.experimental.pallas import tpu_sc as plsc

_NQ = 2048
_K = 3
_BIG = 1e30


def _knn_body(q_ref, kxyzT_ref, bidx_ref, idx_ref, w_ref, *, half):
    q = q_ref[...]
    bq = q.shape[0]
    m = kxyzT_ref.shape[1]
    kT = kxyzT_ref[...]
    cross = jax.lax.dot_general(q, kT, (((1,), (0,)), ((), ())),
                                preferred_element_type=jnp.float32)
    k2 = jnp.sum(kT * kT, axis=0, keepdims=True)
    q2 = jnp.sum(q * q, axis=1, keepdims=True)
    acc = (q2 + k2) - 2.0 * cross
    bid = bidx_ref[...]
    d2 = jnp.where(bid == 0, acc[:, :half], acc[:, half:])
    colid = jax.lax.broadcasted_iota(jnp.int32, (bq, half), 1)
    idxs, vals = [], []
    for k in range(_K):
        mv = jnp.min(d2, axis=1, keepdims=True)
        am = jnp.min(jnp.where(d2 == mv, colid, half), axis=1, keepdims=True)
        idxs.append(am)
        vals.append(mv)
        if k < _K - 1:
            d2 = jnp.where(colid == am, _BIG, d2)
    idx = jnp.concatenate(idxs, axis=1)
    val = jnp.concatenate(vals, axis=1)
    dist = jnp.maximum(val, 1e-10)
    recip = 1.0 / (dist + 1e-8)
    w = recip / jnp.sum(recip, axis=1, keepdims=True)
    idx_ref[...] = idx + bid * half
    w_ref[...] = w


_KNN_BQ = {8192: 128, 2048: 256, 512: 512, 128: 2048, 32: 2048}


def _knn(q, kxyzT, bidx2d, half):
    import functools
    m = kxyzT.shape[1]
    bq = _KNN_BQ[half]
    grid = (_NQ // bq,)
    return pl.pallas_call(
        functools.partial(_knn_body, half=half),
        grid=grid,
        in_specs=[
            pl.BlockSpec((bq, 3), lambda i: (i, 0)),
            pl.BlockSpec((3, m), lambda i: (0, 0)),
            pl.BlockSpec((bq, 1), lambda i: (i, 0)),
        ],
        out_specs=[
            pl.BlockSpec((bq, _K), lambda i: (i, 0)),
            pl.BlockSpec((bq, _K), lambda i: (i, 0)),
        ],
        out_shape=[
            jax.ShapeDtypeStruct((_NQ, _K), jnp.int32),
            jax.ShapeDtypeStruct((_NQ, _K), jnp.float32),
        ],
    )(q, kxyzT, bidx2d)


def _sc_gather(data, idx_flat, csub):
    rows, c = data.shape
    chunks = c // csub
    if chunks > 1:
        data = data.reshape(rows * chunks, csub)
        idx_flat = (idx_flat[0][:, None] * chunks
                    + jnp.arange(chunks, dtype=jnp.int32)).reshape(1, -1)
    window = 128
    num = idx_flat.shape[1]
    c = csub
    mesh = plsc.VectorSubcoreMesh(core_axis_name="c", subcore_axis_name="s")

    @pl.kernel(out_type=jax.ShapeDtypeStruct((num, c), data.dtype), mesh=mesh)
    def k(x_hbm, i_hbm, o_hbm):
        def body(i_vmem, o_vmem):
            pltpu.sync_copy(x_hbm.at[i_vmem.at[0]], o_vmem)

        pltpu.emit_pipeline(
            body,
            grid=(num // window,),
            in_specs=[pl.BlockSpec((1, window), index_map=lambda i: (0, i))],
            out_specs=[pl.BlockSpec((window, c), index_map=lambda i: (i, 0))],
            core_axis_name=("c", "s"),
            dimension_semantics=(pltpu.PARALLEL,),
        )(i_hbm, o_hbm)

    out = k(data, idx_flat)
    if chunks > 1:
        out = out.reshape(num // chunks, chunks * csub)
    return out


def _mlp_body(*refs):
    (w1r, w2r, w3r, w4r, w5r,
     g10, g11, g12, g20, g21, g22,
     i3, i4, i5, t3, t4, t5,
     w1cat, m2, m3, m4, b4r, out_ref) = refs
    wrs = (w1r, w2r, w3r, w4r, w5r)
    grs = ((g10, g11, g12), (g20, g21, g22))
    dn = (((1,), (1,)), ((), ()))
    dn0 = (((1,), (0,)), ((), ()))
    xs = []
    for i in range(2):
        wv = wrs[i][...]
        xs.append(grs[i][0][...].astype(jnp.float32) * wv[:, 0:1]
                  + grs[i][1][...].astype(jnp.float32) * wv[:, 1:2]
                  + grs[i][2][...].astype(jnp.float32) * wv[:, 2:3])
    bq = wrs[0].shape[0]
    for j, (ir, tr) in enumerate(((i3, t3), (i4, t4), (i5, t5))):
        rows = tr.shape[0]
        idxv = ir[...]
        wv = wrs[2 + j][...]
        colid = jax.lax.broadcasted_iota(jnp.int32, (bq, rows), 1)
        s = jnp.where(colid == idxv[:, 0:1], wv[:, 0:1], 0.0)
        s = s + jnp.where(colid == idxv[:, 1:2], wv[:, 1:2], 0.0)
        s = s + jnp.where(colid == idxv[:, 2:3], wv[:, 2:3], 0.0)
        xs.append(jax.lax.dot_general(s, tr[...], dn0,
                                      precision=jax.lax.Precision.HIGHEST,
                                      preferred_element_type=jnp.float32))
    x = jnp.concatenate(xs, axis=1)
    h = jax.lax.dot_general(x, w1cat[...], dn,
                            preferred_element_type=jnp.float32)
    h = jnp.maximum(h, 0.0)
    h = jnp.maximum(jax.lax.dot_general(h, m2[...], dn,
                                        preferred_element_type=jnp.float32), 0.0)
    h = jnp.maximum(jax.lax.dot_general(h, m3[...], dn,
                                        preferred_element_type=jnp.float32), 0.0)
    out_ref[...] = (jax.lax.dot_general(h, m4[...], dn,
                                        preferred_element_type=jnp.float32)
                    + b4r[...])


def _mlp(ws, gs, idxs, tabs, w1cat, w2, w3, w4, b4_2d, n_classes):
    bq = 256
    grid = (_NQ // bq,)
    in_arrays = list(ws)
    in_specs = [pl.BlockSpec((bq, _K), lambda i: (i, 0)) for _ in ws]
    for trip in gs:
        for g in trip:
            in_arrays.append(g)
            in_specs.append(pl.BlockSpec((bq, g.shape[1]), lambda i: (i, 0)))
    for ix in idxs:
        in_arrays.append(ix)
        in_specs.append(pl.BlockSpec((bq, _K), lambda i: (i, 0)))
    for t in tabs:
        in_arrays.append(t)
        in_specs.append(pl.BlockSpec(t.shape, lambda i: (0, 0)))
    for mmat in (w1cat, w2, w3, w4, b4_2d):
        in_arrays.append(mmat)
        in_specs.append(pl.BlockSpec(mmat.shape, lambda i: (0, 0)))
    return pl.pallas_call(
        _mlp_body,
        grid=grid,
        in_specs=in_specs,
        out_specs=pl.BlockSpec((bq, n_classes), lambda i: (i, 0)),
        out_shape=jax.ShapeDtypeStruct((_NQ, n_classes), jnp.float32),
    )(*in_arrays)


def kernel(weakly_points, res1_xyz, res1_features, res2_xyz, res2_features,
           res3_xyz, res3_features, res4_xyz, res4_features, res5_xyz,
           res5_features, batch_inds, W1, W2, W3, W4, b4):
    q = weakly_points
    bidx2d = batch_inds.reshape(_NQ, 1)
    stages = [(res1_xyz, res1_features), (res2_xyz, res2_features),
              (res3_xyz, res3_features), (res4_xyz, res4_features),
              (res5_xyz, res5_features)]
    ws, gs, idxs, tabs, w1_parts = [], [], [], [], []
    off = 0
    for si, (xyz, feat) in enumerate(stages):
        b, n, _ = xyz.shape
        c = feat.shape[1]
        kxyzT = xyz.reshape(b * n, 3).T
        idx, w = _knn(q, kxyzT, bidx2d, half=n)
        ws.append(w)
        featT = feat.transpose(0, 2, 1).reshape(b * n, c)
        cp = -(-c // 128) * 128
        if cp != c:
            featT = jnp.pad(featT, ((0, 0), (0, cp - c)))
        if si < 2:
            idx_flat = idx.T.reshape(1, _K * _NQ)
            gath = _sc_gather(featT, idx_flat, cp)
            g3 = gath.reshape(_K, _NQ, cp)
            gs.append((g3[0], g3[1], g3[2]))
        else:
            idxs.append(idx)
            tabs.append(featT)
        w1p = W1[:, off:off + c]
        if cp != c:
            w1p = jnp.pad(w1p, ((0, 0), (0, cp - c)))
        w1_parts.append(w1p)
        off += c
    n_classes = W4.shape[0]
    b4_2d = b4.reshape(1, n_classes)
    w1cat = jnp.concatenate(w1_parts, axis=1)
    return _mlp(ws, gs, idxs, tabs, w1cat, W2, W3, W4, b4_2d, n_classes)


# speedup vs baseline: 1.6295x; 1.0655x over previous
import jax
import jax.numpy as jnp
from jax.experimental import pallas as pl
from jax.experimental.pallas import tpu as pltpu
from jax.experimental.pallas import tpu_sc as plsc

_NQ = 2048
_K = 3
_BIG = 1e30


def _knn_body(q_ref, kxyzT_ref, bidx_ref, idx_ref, w_ref, *, half):
    q = q_ref[...]
    bq = q.shape[0]
    m = kxyzT_ref.shape[1]
    kT = kxyzT_ref[...]
    cross = jax.lax.dot_general(q, kT, (((1,), (0,)), ((), ())),
                                preferred_element_type=jnp.float32)
    k2 = jnp.sum(kT * kT, axis=0, keepdims=True)
    q2 = jnp.sum(q * q, axis=1, keepdims=True)
    acc = (q2 + k2) - 2.0 * cross
    bid = bidx_ref[...]
    d2 = jnp.where(bid == 0, acc[:, :half], acc[:, half:])
    colid = jax.lax.broadcasted_iota(jnp.int32, (bq, half), 1)
    idxs, vals = [], []
    for k in range(_K):
        mv = jnp.min(d2, axis=1, keepdims=True)
        am = jnp.min(jnp.where(d2 == mv, colid, half), axis=1, keepdims=True)
        idxs.append(am)
        vals.append(mv)
        if k < _K - 1:
            d2 = jnp.where(colid == am, _BIG, d2)
    idx = jnp.concatenate(idxs, axis=1)
    val = jnp.concatenate(vals, axis=1)
    dist = jnp.maximum(val, 1e-10)
    recip = 1.0 / (dist + 1e-8)
    w = recip / jnp.sum(recip, axis=1, keepdims=True)
    idx_ref[...] = idx + bid * half
    w_ref[...] = w


_KNN_BQ = {8192: 256, 2048: 256, 512: 512, 128: 2048, 32: 2048}


def _knn(q, kxyzT, bidx2d, half):
    import functools
    m = kxyzT.shape[1]
    bq = _KNN_BQ[half]
    grid = (_NQ // bq,)
    return pl.pallas_call(
        functools.partial(_knn_body, half=half),
        grid=grid,
        in_specs=[
            pl.BlockSpec((bq, 3), lambda i: (i, 0)),
            pl.BlockSpec((3, m), lambda i: (0, 0)),
            pl.BlockSpec((bq, 1), lambda i: (i, 0)),
        ],
        out_specs=[
            pl.BlockSpec((bq, _K), lambda i: (i, 0)),
            pl.BlockSpec((bq, _K), lambda i: (i, 0)),
        ],
        out_shape=[
            jax.ShapeDtypeStruct((_NQ, _K), jnp.int32),
            jax.ShapeDtypeStruct((_NQ, _K), jnp.float32),
        ],
    )(q, kxyzT, bidx2d)


def _sc_gather(data, idx_flat, csub):
    rows, c = data.shape
    chunks = c // csub
    if chunks > 1:
        data = data.reshape(rows * chunks, csub)
        idx_flat = (idx_flat[0][:, None] * chunks
                    + jnp.arange(chunks, dtype=jnp.int32)).reshape(1, -1)
    window = 128
    num = idx_flat.shape[1]
    c = csub
    mesh = plsc.VectorSubcoreMesh(core_axis_name="c", subcore_axis_name="s")

    @pl.kernel(out_type=jax.ShapeDtypeStruct((num, c), data.dtype), mesh=mesh)
    def k(x_hbm, i_hbm, o_hbm):
        def body(i_vmem, o_vmem):
            pltpu.sync_copy(x_hbm.at[i_vmem.at[0]], o_vmem)

        pltpu.emit_pipeline(
            body,
            grid=(num // window,),
            in_specs=[pl.BlockSpec((1, window), index_map=lambda i: (0, i))],
            out_specs=[pl.BlockSpec((window, c), index_map=lambda i: (i, 0))],
            core_axis_name=("c", "s"),
            dimension_semantics=(pltpu.PARALLEL,),
        )(i_hbm, o_hbm)

    out = k(data, idx_flat)
    if chunks > 1:
        out = out.reshape(num // chunks, chunks * csub)
    return out


def _mlp_body(*refs):
    (w1r, w2r, w3r, w4r, w5r,
     g10, g11, g12, g20, g21, g22,
     i3, i4, i5, t3h, t3l, t4h, t4l, t5h, t5l,
     w1cat, m2, m3, m4, b4r, out_ref) = refs
    wrs = (w1r, w2r, w3r, w4r, w5r)
    grs = ((g10, g11, g12), (g20, g21, g22))
    dn = (((1,), (1,)), ((), ()))
    dn0 = (((1,), (0,)), ((), ()))
    xs = []
    for i in range(2):
        wv = wrs[i][...]
        xs.append(grs[i][0][...].astype(jnp.float32) * wv[:, 0:1]
                  + grs[i][1][...].astype(jnp.float32) * wv[:, 1:2]
                  + grs[i][2][...].astype(jnp.float32) * wv[:, 2:3])
    bq = wrs[0].shape[0]
    for j, (ir, trh, trl) in enumerate(((i3, t3h, t3l), (i4, t4h, t4l),
                                        (i5, t5h, t5l))):
        rows = trh.shape[0]
        idxv = ir[...]
        wv = wrs[2 + j][...]
        colid = jax.lax.broadcasted_iota(jnp.int32, (bq, rows), 1)
        s = jnp.where(colid == idxv[:, 0:1], wv[:, 0:1], 0.0)
        s = s + jnp.where(colid == idxv[:, 1:2], wv[:, 1:2], 0.0)
        s = s + jnp.where(colid == idxv[:, 2:3], wv[:, 2:3], 0.0)
        sh = s.astype(jnp.bfloat16).astype(jnp.float32)
        sl = s - sh
        x3 = jax.lax.dot_general(sh, trh[...], dn0,
                                 preferred_element_type=jnp.float32)
        x3 = x3 + jax.lax.dot_general(sh, trl[...], dn0,
                                      preferred_element_type=jnp.float32)
        x3 = x3 + jax.lax.dot_general(sl, trh[...], dn0,
                                      preferred_element_type=jnp.float32)
        xs.append(x3)
    x = jnp.concatenate(xs, axis=1)
    h = jax.lax.dot_general(x, w1cat[...], dn,
                            preferred_element_type=jnp.float32)
    h = jnp.maximum(h, 0.0)
    h = jnp.maximum(jax.lax.dot_general(h, m2[...], dn,
                                        preferred_element_type=jnp.float32), 0.0)
    h = jnp.maximum(jax.lax.dot_general(h, m3[...], dn,
                                        preferred_element_type=jnp.float32), 0.0)
    out_ref[...] = (jax.lax.dot_general(h, m4[...], dn,
                                        preferred_element_type=jnp.float32)
                    + b4r[...])


def _mlp(ws, gs, idxs, tabs, w1cat, w2, w3, w4, b4_2d, n_classes):
    bq = 256
    grid = (_NQ // bq,)
    in_arrays = list(ws)
    in_specs = [pl.BlockSpec((bq, _K), lambda i: (i, 0)) for _ in ws]
    for trip in gs:
        for g in trip:
            in_arrays.append(g)
            in_specs.append(pl.BlockSpec((bq, g.shape[1]), lambda i: (i, 0)))
    for ix in idxs:
        in_arrays.append(ix)
        in_specs.append(pl.BlockSpec((bq, _K), lambda i: (i, 0)))
    for t in tabs:
        th = t.astype(jnp.bfloat16).astype(jnp.float32)
        for part in (th, t - th):
            in_arrays.append(part)
            in_specs.append(pl.BlockSpec(t.shape, lambda i: (0, 0)))
    for mmat in (w1cat, w2, w3, w4, b4_2d):
        in_arrays.append(mmat)
        in_specs.append(pl.BlockSpec(mmat.shape, lambda i: (0, 0)))
    return pl.pallas_call(
        _mlp_body,
        grid=grid,
        in_specs=in_specs,
        out_specs=pl.BlockSpec((bq, n_classes), lambda i: (i, 0)),
        out_shape=jax.ShapeDtypeStruct((_NQ, n_classes), jnp.float32),
    )(*in_arrays)


def kernel(weakly_points, res1_xyz, res1_features, res2_xyz, res2_features,
           res3_xyz, res3_features, res4_xyz, res4_features, res5_xyz,
           res5_features, batch_inds, W1, W2, W3, W4, b4):
    q = weakly_points
    bidx2d = batch_inds.reshape(_NQ, 1)
    stages = [(res1_xyz, res1_features), (res2_xyz, res2_features),
              (res3_xyz, res3_features), (res4_xyz, res4_features),
              (res5_xyz, res5_features)]
    ws, gs, idxs, tabs, w1_parts = [], [], [], [], []
    off = 0
    for si, (xyz, feat) in enumerate(stages):
        b, n, _ = xyz.shape
        c = feat.shape[1]
        kxyzT = xyz.reshape(b * n, 3).T
        idx, w = _knn(q, kxyzT, bidx2d, half=n)
        ws.append(w)
        featT = feat.transpose(0, 2, 1).reshape(b * n, c)
        cp = -(-c // 128) * 128
        if cp != c:
            featT = jnp.pad(featT, ((0, 0), (0, cp - c)))
        if si < 2:
            idx_flat = idx.T.reshape(1, _K * _NQ)
            gath = _sc_gather(featT, idx_flat, cp)
            g3 = gath.reshape(_K, _NQ, cp)
            gs.append((g3[0], g3[1], g3[2]))
        else:
            idxs.append(idx)
            tabs.append(featT)
        w1p = W1[:, off:off + c]
        if cp != c:
            w1p = jnp.pad(w1p, ((0, 0), (0, cp - c)))
        w1_parts.append(w1p)
        off += c
    n_classes = W4.shape[0]
    b4_2d = b4.reshape(1, n_classes)
    w1cat = jnp.concatenate(w1_parts, axis=1)
    return _mlp(ws, gs, idxs, tabs, w1cat, W2, W3, W4, b4_2d, n_classes)


# speedup vs baseline: 1.8751x; 1.1507x over previous
import jax
import jax.numpy as jnp
from jax.experimental import pallas as pl
from jax.experimental.pallas import tpu as pltpu
from jax.experimental.pallas import tpu_sc as plsc

_NQ = 2048
_K = 3
_BIG = 1e30


def _knn_body(q_ref, kxyzT_ref, bidx_ref, idx_ref, w_ref, *, half):
    q = q_ref[...]
    bq = q.shape[0]
    m = kxyzT_ref.shape[1]
    kT = kxyzT_ref[...]
    cross = jax.lax.dot_general(q, kT, (((1,), (0,)), ((), ())),
                                preferred_element_type=jnp.float32)
    k2 = jnp.sum(kT * kT, axis=0, keepdims=True)
    q2 = jnp.sum(q * q, axis=1, keepdims=True)
    acc = (q2 + k2) - 2.0 * cross
    bid = bidx_ref[...]
    d2 = jnp.where(bid == 0, acc[:, :half], acc[:, half:])
    colid = jax.lax.broadcasted_iota(jnp.int32, (bq, half), 1)
    idxs, vals = [], []
    for k in range(_K):
        mv = jnp.min(d2, axis=1, keepdims=True)
        am = jnp.min(jnp.where(d2 == mv, colid, half), axis=1, keepdims=True)
        idxs.append(am)
        vals.append(mv)
        if k < _K - 1:
            d2 = jnp.where(colid == am, _BIG, d2)
    idx = jnp.concatenate(idxs, axis=1)
    val = jnp.concatenate(vals, axis=1)
    dist = jnp.maximum(val, 1e-10)
    recip = 1.0 / (dist + 1e-8)
    w = recip / jnp.sum(recip, axis=1, keepdims=True)
    idx_ref[...] = idx + bid * half
    w_ref[...] = w


_KNN_BQ = {8192: 256, 2048: 512, 512: 512, 128: 2048, 32: 2048}


def _knn(q, kxyzT, bidx2d, half):
    import functools
    m = kxyzT.shape[1]
    bq = _KNN_BQ[half]
    grid = (_NQ // bq,)
    return pl.pallas_call(
        functools.partial(_knn_body, half=half),
        grid=grid,
        in_specs=[
            pl.BlockSpec((bq, 3), lambda i: (i, 0)),
            pl.BlockSpec((3, m), lambda i: (0, 0)),
            pl.BlockSpec((bq, 1), lambda i: (i, 0)),
        ],
        out_specs=[
            pl.BlockSpec((bq, _K), lambda i: (i, 0)),
            pl.BlockSpec((bq, _K), lambda i: (i, 0)),
        ],
        out_shape=[
            jax.ShapeDtypeStruct((_NQ, _K), jnp.int32),
            jax.ShapeDtypeStruct((_NQ, _K), jnp.float32),
        ],
    )(q, kxyzT, bidx2d)


def _sc_gather(data, idx_flat, csub):
    rows, c = data.shape
    chunks = c // csub
    if chunks > 1:
        data = data.reshape(rows * chunks, csub)
        idx_flat = (idx_flat[0][:, None] * chunks
                    + jnp.arange(chunks, dtype=jnp.int32)).reshape(1, -1)
    window = 128
    num = idx_flat.shape[1]
    c = csub
    mesh = plsc.VectorSubcoreMesh(core_axis_name="c", subcore_axis_name="s")

    @pl.kernel(out_type=jax.ShapeDtypeStruct((num, c), data.dtype), mesh=mesh)
    def k(x_hbm, i_hbm, o_hbm):
        def body(i_vmem, o_vmem):
            pltpu.sync_copy(x_hbm.at[i_vmem.at[0]], o_vmem)

        pltpu.emit_pipeline(
            body,
            grid=(num // window,),
            in_specs=[pl.BlockSpec((1, window), index_map=lambda i: (0, i))],
            out_specs=[pl.BlockSpec((window, c), index_map=lambda i: (i, 0))],
            core_axis_name=("c", "s"),
            dimension_semantics=(pltpu.PARALLEL,),
        )(i_hbm, o_hbm)

    out = k(data, idx_flat)
    if chunks > 1:
        out = out.reshape(num // chunks, chunks * csub)
    return out


def _mlp_body(*refs):
    (w1r, w2r, w3r, w4r, w5r,
     g10, g11, g12, g20, g21, g22,
     i3, i4, i5, t3h, t3l, t4h, t4l, t5h, t5l,
     w1r_full, m2, m3, m4, b4r, out_ref) = refs
    wrs = (w1r, w2r, w3r, w4r, w5r)
    grs = ((g10, g11, g12), (g20, g21, g22))
    dn = (((1,), (1,)), ((), ()))
    dn0 = (((1,), (0,)), ((), ()))
    xs = []
    for i, c in zip(range(2), (144, 288)):
        wv = wrs[i][...]
        xi = (grs[i][0][...].astype(jnp.float32) * wv[:, 0:1]
              + grs[i][1][...].astype(jnp.float32) * wv[:, 1:2]
              + grs[i][2][...].astype(jnp.float32) * wv[:, 2:3])
        xs.append(xi[:, :c])
    bq = wrs[0].shape[0]
    for j, (ir, trh, trl) in enumerate(((i3, t3h, t3l), (i4, t4h, t4l),
                                        (i5, t5h, t5l))):
        rows = trh.shape[0]
        idxv = ir[...]
        wv = wrs[2 + j][...]
        colid = jax.lax.broadcasted_iota(jnp.int32, (bq, rows), 1)
        s = jnp.where(colid == idxv[:, 0:1], wv[:, 0:1], 0.0)
        s = s + jnp.where(colid == idxv[:, 1:2], wv[:, 1:2], 0.0)
        s = s + jnp.where(colid == idxv[:, 2:3], wv[:, 2:3], 0.0)
        sh = s.astype(jnp.bfloat16).astype(jnp.float32)
        sl = s - sh
        x3 = jax.lax.dot_general(sh, trh[...], dn0,
                                 preferred_element_type=jnp.float32)
        x3 = x3 + jax.lax.dot_general(sh, trl[...], dn0,
                                      preferred_element_type=jnp.float32)
        x3 = x3 + jax.lax.dot_general(sl, trh[...], dn0,
                                      preferred_element_type=jnp.float32)
        xs.append(x3)
    x = jnp.concatenate(xs, axis=1)
    h = jax.lax.dot_general(x, w1r_full[...], dn,
                            preferred_element_type=jnp.float32)
    h = jnp.maximum(h, 0.0)
    h = jnp.maximum(jax.lax.dot_general(h, m2[...], dn,
                                        preferred_element_type=jnp.float32), 0.0)
    h = jnp.maximum(jax.lax.dot_general(h, m3[...], dn,
                                        preferred_element_type=jnp.float32), 0.0)
    out_ref[...] = (jax.lax.dot_general(h, m4[...], dn,
                                        preferred_element_type=jnp.float32)
                    + b4r[...])


def _mlp(ws, gs, idxs, tabs, w1, w2, w3, w4, b4_2d, n_classes):
    bq = 256
    grid = (_NQ // bq,)
    in_arrays = list(ws)
    in_specs = [pl.BlockSpec((bq, _K), lambda i: (i, 0)) for _ in ws]
    for trip in gs:
        for g in trip:
            in_arrays.append(g)
            in_specs.append(pl.BlockSpec((bq, g.shape[1]), lambda i: (i, 0)))
    for ix in idxs:
        in_arrays.append(ix)
        in_specs.append(pl.BlockSpec((bq, _K), lambda i: (i, 0)))
    for t in tabs:
        th = t.astype(jnp.bfloat16).astype(jnp.float32)
        for part in (th, t - th):
            in_arrays.append(part)
            in_specs.append(pl.BlockSpec(t.shape, lambda i: (0, 0)))
    for mmat in (w1, w2, w3, w4, b4_2d):
        in_arrays.append(mmat)
        in_specs.append(pl.BlockSpec(mmat.shape, lambda i: (0, 0)))
    return pl.pallas_call(
        _mlp_body,
        grid=grid,
        in_specs=in_specs,
        out_specs=pl.BlockSpec((bq, n_classes), lambda i: (i, 0)),
        out_shape=jax.ShapeDtypeStruct((_NQ, n_classes), jnp.float32),
    )(*in_arrays)


def kernel(weakly_points, res1_xyz, res1_features, res2_xyz, res2_features,
           res3_xyz, res3_features, res4_xyz, res4_features, res5_xyz,
           res5_features, batch_inds, W1, W2, W3, W4, b4):
    q = weakly_points
    bidx2d = batch_inds.reshape(_NQ, 1)
    stages = [(res1_xyz, res1_features), (res2_xyz, res2_features),
              (res3_xyz, res3_features), (res4_xyz, res4_features),
              (res5_xyz, res5_features)]
    ws, gs, idxs, tabs = [], [], [], []
    for si, (xyz, feat) in enumerate(stages):
        b, n, _ = xyz.shape
        c = feat.shape[1]
        kxyzT = xyz.reshape(b * n, 3).T
        idx, w = _knn(q, kxyzT, bidx2d, half=n)
        ws.append(w)
        featT = feat.transpose(0, 2, 1).reshape(b * n, c)
        if si < 2:
            cp = -(-c // 128) * 128
            featT = jnp.pad(featT, ((0, 0), (0, cp - c)))
            idx_flat = idx.T.reshape(1, _K * _NQ)
            gath = _sc_gather(featT, idx_flat, cp)
            g3 = gath.reshape(_K, _NQ, cp)
            gs.append((g3[0], g3[1], g3[2]))
        else:
            idxs.append(idx)
            tabs.append(featT)
    n_classes = W4.shape[0]
    b4_2d = b4.reshape(1, n_classes)
    return _mlp(ws, gs, idxs, tabs, W1, W2, W3, W4, b4_2d, n_classes)


# speedup vs baseline: 1.8842x; 1.0048x over previous
import functools

import jax
import jax.numpy as jnp
from jax.experimental import pallas as pl
from jax.experimental.pallas import tpu as pltpu
from jax.experimental.pallas import tpu_sc as plsc

_NQ = 2048
_K = 3
_BIG = 1e30


def _knn_body(q_ref, kxyzT_ref, bidx_ref, idx_ref, w_ref, *, half):
    q = q_ref[...]
    bq = q.shape[0]
    m = kxyzT_ref.shape[1]
    kT = kxyzT_ref[...]
    cross = jax.lax.dot_general(q, kT, (((1,), (0,)), ((), ())),
                                preferred_element_type=jnp.float32)
    k2 = jnp.sum(kT * kT, axis=0, keepdims=True)
    q2 = jnp.sum(q * q, axis=1, keepdims=True)
    acc = (q2 + k2) - 2.0 * cross
    bid = bidx_ref[...]
    d2 = jnp.where(bid == 0, acc[:, :half], acc[:, half:])
    colid = jax.lax.broadcasted_iota(jnp.int32, (bq, half), 1)
    idxs, vals = [], []
    for k in range(_K):
        mv = jnp.min(d2, axis=1, keepdims=True)
        am = jnp.min(jnp.where(d2 == mv, colid, half), axis=1, keepdims=True)
        idxs.append(am)
        vals.append(mv)
        if k < _K - 1:
            d2 = jnp.where(colid == am, _BIG, d2)
    idx = jnp.concatenate(idxs, axis=1)
    val = jnp.concatenate(vals, axis=1)
    dist = jnp.maximum(val, 1e-10)
    recip = 1.0 / (dist + 1e-8)
    w = recip / jnp.sum(recip, axis=1, keepdims=True)
    idx_ref[...] = idx + bid * half
    w_ref[...] = w


_KNN_BQ = {8192: 256, 2048: 512, 512: 512, 128: 2048, 32: 2048}


def _knn(q, kxyzT, bidx2d, half):
    m = kxyzT.shape[1]
    bq = _KNN_BQ[half]
    grid = (_NQ // bq,)
    return pl.pallas_call(
        functools.partial(_knn_body, half=half),
        grid=grid,
        in_specs=[
            pl.BlockSpec((bq, 3), lambda i: (i, 0)),
            pl.BlockSpec((3, m), lambda i: (0, 0)),
            pl.BlockSpec((bq, 1), lambda i: (i, 0)),
        ],
        out_specs=[
            pl.BlockSpec((bq, _K), lambda i: (i, 0)),
            pl.BlockSpec((bq, _K), lambda i: (i, 0)),
        ],
        out_shape=[
            jax.ShapeDtypeStruct((_NQ, _K), jnp.int32),
            jax.ShapeDtypeStruct((_NQ, _K), jnp.float32),
        ],
    )(q, kxyzT, bidx2d)


def _sc_gather(data, idx_flat, csub):
    rows, c = data.shape
    chunks = c // csub
    if chunks > 1:
        data = data.reshape(rows * chunks, csub)
        idx_flat = (idx_flat[0][:, None] * chunks
                    + jnp.arange(chunks, dtype=jnp.int32)).reshape(1, -1)
    window = 128
    num = idx_flat.shape[1]
    c = csub
    mesh = plsc.VectorSubcoreMesh(core_axis_name="c", subcore_axis_name="s")

    @pl.kernel(out_type=jax.ShapeDtypeStruct((num, c), data.dtype), mesh=mesh)
    def k(x_hbm, i_hbm, o_hbm):
        def body(i_vmem, o_vmem):
            pltpu.sync_copy(x_hbm.at[i_vmem.at[0]], o_vmem)

        pltpu.emit_pipeline(
            body,
            grid=(num // window,),
            in_specs=[pl.BlockSpec((1, window), index_map=lambda i: (0, i))],
            out_specs=[pl.BlockSpec((window, c), index_map=lambda i: (i, 0))],
            core_axis_name=("c", "s"),
            dimension_semantics=(pltpu.PARALLEL,),
        )(i_hbm, o_hbm)

    out = k(data, idx_flat)
    if chunks > 1:
        out = out.reshape(num // chunks, chunks * csub)
    return out


def _mlp_body(*refs):
    (w1r, w2r, w3r, w4r, w5r,
     g10, g11, g12, g20, g21, g22,
     i3, i4, i5, t3h, t3l, t4h, t4l, t5h, t5l,
     w1r_full, m2, m3, m4, b4r, out_ref) = refs
    wrs = (w1r, w2r, w3r, w4r, w5r)
    grs = ((g10, g11, g12), (g20, g21, g22))
    dn = (((1,), (1,)), ((), ()))
    dn0 = (((1,), (0,)), ((), ()))
    xs = []
    for i, c in zip(range(2), (144, 288)):
        wv = wrs[i][...]
        xi = (grs[i][0][...].astype(jnp.float32) * wv[:, 0:1]
              + grs[i][1][...].astype(jnp.float32) * wv[:, 1:2]
              + grs[i][2][...].astype(jnp.float32) * wv[:, 2:3])
        xs.append(xi[:, :c])
    bq = wrs[0].shape[0]
    for j, (ir, trh, trl) in enumerate(((i3, t3h, t3l), (i4, t4h, t4l),
                                        (i5, t5h, t5l))):
        rows = trh.shape[0]
        idxv = ir[...]
        wv = wrs[2 + j][...]
        colid = jax.lax.broadcasted_iota(jnp.int32, (bq, rows), 1)
        s = jnp.where(colid == idxv[:, 0:1], wv[:, 0:1], 0.0)
        s = s + jnp.where(colid == idxv[:, 1:2], wv[:, 1:2], 0.0)
        s = s + jnp.where(colid == idxv[:, 2:3], wv[:, 2:3], 0.0)
        sh = s.astype(jnp.bfloat16).astype(jnp.float32)
        sl = s - sh
        x3 = jax.lax.dot_general(sh, trh[...], dn0,
                                 preferred_element_type=jnp.float32)
        x3 = x3 + jax.lax.dot_general(sh, trl[...], dn0,
                                      preferred_element_type=jnp.float32)
        x3 = x3 + jax.lax.dot_general(sl, trh[...], dn0,
                                      preferred_element_type=jnp.float32)
        xs.append(x3)
    x = jnp.concatenate(xs, axis=1)
    h = jax.lax.dot_general(x, w1r_full[...], dn,
                            preferred_element_type=jnp.float32)
    h = jnp.maximum(h, 0.0)
    h = jnp.maximum(jax.lax.dot_general(h, m2[...], dn,
                                        preferred_element_type=jnp.float32), 0.0)
    h = jnp.maximum(jax.lax.dot_general(h, m3[...], dn,
                                        preferred_element_type=jnp.float32), 0.0)
    out_ref[...] = (jax.lax.dot_general(h, m4[...], dn,
                                        preferred_element_type=jnp.float32)
                    + b4r[...])


def _mlp(ws, gs, idxs, tabs, w1, w2, w3, w4, b4_2d, n_classes):
    bq = 512
    grid = (_NQ // bq,)
    in_arrays = list(ws)
    in_specs = [pl.BlockSpec((bq, _K), lambda i: (i, 0)) for _ in ws]
    for trip in gs:
        for g in trip:
            in_arrays.append(g)
            in_specs.append(pl.BlockSpec((bq, g.shape[1]), lambda i: (i, 0)))
    for ix in idxs:
        in_arrays.append(ix)
        in_specs.append(pl.BlockSpec((bq, _K), lambda i: (i, 0)))
    for t in tabs:
        th = t.astype(jnp.bfloat16).astype(jnp.float32)
        for part in (th, t - th):
            in_arrays.append(part)
            in_specs.append(pl.BlockSpec(t.shape, lambda i: (0, 0)))
    for mmat in (w1, w2, w3, w4, b4_2d):
        in_arrays.append(mmat)
        in_specs.append(pl.BlockSpec(mmat.shape, lambda i: (0, 0)))
    return pl.pallas_call(
        _mlp_body,
        grid=grid,
        in_specs=in_specs,
        out_specs=pl.BlockSpec((bq, n_classes), lambda i: (i, 0)),
        out_shape=jax.ShapeDtypeStruct((_NQ, n_classes), jnp.float32),
    )(*in_arrays)


def kernel(weakly_points, res1_xyz, res1_features, res2_xyz, res2_features,
           res3_xyz, res3_features, res4_xyz, res4_features, res5_xyz,
           res5_features, batch_inds, W1, W2, W3, W4, b4):
    q = weakly_points
    bidx2d = batch_inds.reshape(_NQ, 1)
    stages = [(res1_xyz, res1_features), (res2_xyz, res2_features),
              (res3_xyz, res3_features), (res4_xyz, res4_features),
              (res5_xyz, res5_features)]
    ws, gs, idxs, tabs = [], [], [], []
    for si, (xyz, feat) in enumerate(stages):
        b, n, _ = xyz.shape
        c = feat.shape[1]
        kxyzT = xyz.reshape(b * n, 3).T
        idx, w = _knn(q, kxyzT, bidx2d, half=n)
        ws.append(w)
        featT = feat.transpose(0, 2, 1).reshape(b * n, c)
        if si < 2:
            cp = -(-c // 128) * 128
            featT = jnp.pad(featT, ((0, 0), (0, cp - c)))
            idx_flat = idx.T.reshape(1, _K * _NQ)
            gath = _sc_gather(featT, idx_flat, cp)
            g3 = gath.reshape(_K, _NQ, cp)
            gs.append((g3[0], g3[1], g3[2]))
        else:
            idxs.append(idx)
            tabs.append(featT)
    n_classes = W4.shape[0]
    b4_2d = b4.reshape(1, n_classes)
    return _mlp(ws, gs, idxs, tabs, W1, W2, W3, W4, b4_2d, n_classes)
